# baseline (speedup 1.0000x reference)
import jax
import jax.numpy as jnp
from jax.experimental import pallas as pl
from jax.experimental.pallas import tpu as pltpu

B, H, S, D, Q = 1, 8, 2048, 128, 16
C = 256
G = S // C


def _body(pos_ref, *refs):
    ins = refs[0:16]
    outs = refs[16:24]
    pos = pos_ref[0]
    i = pl.program_id(0)
    base = i * C

    for l in range(4):
        ck, sk, cv, sv = ins[4 * l], ins[4 * l + 1], ins[4 * l + 2], ins[4 * l + 3]
        ko, vo = outs[2 * l], outs[2 * l + 1]

        ko[...] = ck[...]
        r0 = pos - base
        for q in range(Q):
            rq = r0 + q

            @pl.when((rq >= 0) & (rq < C))
            def _(l=l, q=q, rq=rq, ko=ko, sk=sk):
                ko[0, :, pl.ds(jnp.clip(rq, 0, C - 1), 1), :] = sk[0, :, pl.ds(q, 1), :]

        vo[...] = cv[...]

        @pl.when((pos < base + C) & (pos + Q > base))
        def _(base=base, sv=sv, cv=cv, vo=vo):
            shift = jnp.mod(pos - base, C)
            padded = jnp.pad(sv[0][...], ((0, 0), (0, 0), (0, C - Q)))
            rolled = pltpu.roll(padded, shift, 2)
            lane_g = jax.lax.broadcasted_iota(jnp.int32, (1, 1, C), 2) + base
            mask = (lane_g >= pos) & (lane_g < pos + Q)
            vo[...] = jnp.where(mask[None], rolled[None], cv[...])


def kernel(input_pos, kv_cache_k_0, kv_slice_k_0, kv_cache_v_0, kv_slice_v_0, kv_cache_k_1, kv_slice_k_1, kv_cache_v_1, kv_slice_v_1, kv_cache_k_2, kv_slice_k_2, kv_cache_v_2, kv_slice_v_2, kv_cache_k_3, kv_slice_k_3, kv_cache_v_3, kv_slice_v_3):
    caches_and_slices = (
        kv_cache_k_0, kv_slice_k_0, kv_cache_v_0, kv_slice_v_0,
        kv_cache_k_1, kv_slice_k_1, kv_cache_v_1, kv_slice_v_1,
        kv_cache_k_2, kv_slice_k_2, kv_cache_v_2, kv_slice_v_2,
        kv_cache_k_3, kv_slice_k_3, kv_cache_v_3, kv_slice_v_3,
    )
    k_shape = jax.ShapeDtypeStruct((B, H, S, D), jnp.float32)
    v_shape = jax.ShapeDtypeStruct((B, H, D, S), jnp.float32)
    out_shape = (k_shape, v_shape) * 4

    k_cache_spec = pl.BlockSpec((B, H, C, D), lambda i, p: (0, 0, i, 0))
    k_slice_spec = pl.BlockSpec((B, H, Q, D), lambda i, p: (0, 0, 0, 0))
    v_cache_spec = pl.BlockSpec((B, H, D, C), lambda i, p: (0, 0, 0, i))
    v_slice_spec = pl.BlockSpec((B, H, D, Q), lambda i, p: (0, 0, 0, 0))

    grid_spec = pltpu.PrefetchScalarGridSpec(
        num_scalar_prefetch=1,
        grid=(G,),
        in_specs=[k_cache_spec, k_slice_spec, v_cache_spec, v_slice_spec] * 4,
        out_specs=[k_cache_spec, v_cache_spec] * 4,
    )

    outs = pl.pallas_call(
        _body,
        grid_spec=grid_spec,
        out_shape=out_shape,
        compiler_params=pltpu.CompilerParams(
            dimension_semantics=("parallel",),
        ),
    )(input_pos.astype(jnp.int32), *caches_and_slices)
    return tuple(outs)


# speedup vs baseline: 1.0044x; 1.0044x over previous
import jax
import jax.numpy as jnp
from jax.experimental import pallas as pl
from jax.experimental.pallas import tpu as pltpu

B, H, S, D, Q = 1, 8, 2048, 128, 16
C = 256
G = S // C


def _body(pos_ref, *refs):
    ins = refs[0:16]
    outs = refs[16:24]
    pos = pos_ref[0]
    i = pl.program_id(0)
    base = i * C

    for l in range(4):
        ck, sk, cv, sv = ins[4 * l], ins[4 * l + 1], ins[4 * l + 2], ins[4 * l + 3]
        ko, vo = outs[2 * l], outs[2 * l + 1]

        ko[...] = ck[...]
        r0 = pos - base
        for q in range(Q):
            rq = r0 + q

            @pl.when((rq >= 0) & (rq < C))
            def _(l=l, q=q, rq=rq, ko=ko, sk=sk):
                ko[0, :, pl.ds(jnp.clip(rq, 0, C - 1), 1), :] = sk[0, :, pl.ds(q, 1), :]

        vo[...] = cv[...]

        @pl.when((pos < base + C) & (pos + Q > base))
        def _(base=base, sv=sv, cv=cv, vo=vo):
            shift = jnp.mod(pos - base, C)
            padded = jnp.pad(sv[0][...], ((0, 0), (0, 0), (0, C - Q)))
            rolled = pltpu.roll(padded, shift, 2)
            lane_g = jax.lax.broadcasted_iota(jnp.int32, (1, 1, C), 2) + base
            mask = (lane_g >= pos) & (lane_g < pos + Q)
            vo[...] = jnp.where(mask[None], rolled[None], cv[...])


def kernel(input_pos, kv_cache_k_0, kv_slice_k_0, kv_cache_v_0, kv_slice_v_0, kv_cache_k_1, kv_slice_k_1, kv_cache_v_1, kv_slice_v_1, kv_cache_k_2, kv_slice_k_2, kv_cache_v_2, kv_slice_v_2, kv_cache_k_3, kv_slice_k_3, kv_cache_v_3, kv_slice_v_3):
    caches_and_slices = (
        kv_cache_k_0, kv_slice_k_0, kv_cache_v_0, kv_slice_v_0,
        kv_cache_k_1, kv_slice_k_1, kv_cache_v_1, kv_slice_v_1,
        kv_cache_k_2, kv_slice_k_2, kv_cache_v_2, kv_slice_v_2,
        kv_cache_k_3, kv_slice_k_3, kv_cache_v_3, kv_slice_v_3,
    )
    k_shape = jax.ShapeDtypeStruct((B, H, S, D), jnp.float32)
    v_shape = jax.ShapeDtypeStruct((B, H, D, S), jnp.float32)
    out_shape = (k_shape, v_shape) * 4

    k_cache_spec = pl.BlockSpec((B, H, C, D), lambda i, p: (0, 0, i, 0))
    k_slice_spec = pl.BlockSpec((B, H, Q, D), lambda i, p: (0, 0, 0, 0))
    v_cache_spec = pl.BlockSpec((B, H, D, C), lambda i, p: (0, 0, 0, i))
    v_slice_spec = pl.BlockSpec((B, H, D, Q), lambda i, p: (0, 0, 0, 0))

    grid_spec = pltpu.PrefetchScalarGridSpec(
        num_scalar_prefetch=1,
        grid=(G,),
        in_specs=[k_cache_spec, k_slice_spec, v_cache_spec, v_slice_spec] * 4,
        out_specs=[k_cache_spec, v_cache_spec] * 4,
    )

    outs = pl.pallas_call(
        _body,
        grid_spec=grid_spec,
        out_shape=out_shape,
        compiler_params=pltpu.CompilerParams(
            dimension_semantics=("arbitrary",),
        ),
    )(input_pos.astype(jnp.int32), *caches_and_slices)
    return tuple(outs)
